# baseline (speedup 1.0000x reference)
import functools

import jax
import jax.numpy as jnp
from jax import lax
from jax.experimental import pallas as pl
from jax.experimental.pallas import tpu as pltpu
from jax.experimental.pallas import tpu_sc as plsc

_B, _L = 4096, 20
_BT = _B * _L
_EMBED = 128
_HALF = _EMBED // 2
_PROJ = 256

_NC, _NS = 2, 16
_NW = _NC * _NS
_NSPLIT = 2
_BTS = _BT // _NSPLIT
_BPW = _BTS // _NW
_CH = 80
_NCHUNK = _BPW // _CH


def _pack_rows(fbuf, bbuf):
    lo_mask = jnp.int32(0xFFFF)
    hi_mask = jnp.int32(-65536)
    def body(r, carry):
        for c4 in range(4):
            a = fbuf[r, pl.ds(32 * c4, 16)]
            b = fbuf[r, pl.ds(32 * c4 + 16, 16)]
            ai = lax.bitcast_convert_type(a, jnp.int32)
            bi = lax.bitcast_convert_type(b, jnp.int32)
            word = jnp.bitwise_or(
                jnp.bitwise_and(lax.shift_right_logical(ai, 16), lo_mask),
                jnp.bitwise_and(bi, hi_mask))
            bbuf[r, pl.ds(16 * c4, 16)] = word
        return carry
    lax.fori_loop(0, _CH, body, 0, unroll=4)


def _sc_gather_body(hotel_hbm, travel_hbm, hid_hbm, tid_hbm,
                    out_h_hbm, out_t_hbm,
                    idx_h, idx_t, hbuf0, hbuf1, tbuf0, tbuf1,
                    hpk0, hpk1, tpk0, tpk1,
                    gsh0, gsh1, gst0, gst1, wsh0, wsh1, wst0, wst1):
    hbufs, tbufs = (hbuf0, hbuf1), (tbuf0, tbuf1)
    hpks, tpks = (hpk0, hpk1), (tpk0, tpk1)
    gsems_h, gsems_t = (gsh0, gsh1), (gst0, gst1)
    wsems_h, wsems_t = (wsh0, wsh1), (wst0, wst1)
    wid = lax.axis_index("s") * _NC + lax.axis_index("c")
    base = wid * _BPW
    pltpu.sync_copy(hid_hbm.at[pl.ds(base, _BPW)], idx_h)
    pltpu.sync_copy(tid_hbm.at[pl.ds(base, _BPW)], idx_t)

    def g_h(c):
        return pltpu.async_copy(
            hotel_hbm.at[idx_h.at[pl.ds(c * _CH, _CH)]],
            hbufs[c % 2], gsems_h[c % 2])

    def g_t(c):
        return pltpu.async_copy(
            travel_hbm.at[idx_t.at[pl.ds(c * _CH, _CH)]],
            tbufs[c % 2], gsems_t[c % 2])

    def w_h(c):
        return pltpu.async_copy(
            hpks[c % 2], out_h_hbm.at[pl.ds(base + c * _CH, _CH)],
            wsems_h[c % 2])

    def w_t(c):
        return pltpu.async_copy(
            tpks[c % 2], out_t_hbm.at[pl.ds(base + c * _CH, _CH)],
            wsems_t[c % 2])

    gh = {0: g_h(0)}
    gt = {0: g_t(0)}
    wh, wt = {}, {}
    for c in range(_NCHUNK):
        if c >= 2:
            wh[c - 2].wait()
            wt[c - 2].wait()
        if c + 1 < _NCHUNK:
            gh[c + 1] = g_h(c + 1)
            gt[c + 1] = g_t(c + 1)
        gh[c].wait()
        _pack_rows(hbufs[c % 2], hpks[c % 2])
        wh[c] = w_h(c)
        gt[c].wait()
        _pack_rows(tbufs[c % 2], tpks[c % 2])
        wt[c] = w_t(c)
    wh[_NCHUNK - 2].wait()
    wt[_NCHUNK - 2].wait()
    wh[_NCHUNK - 1].wait()
    wt[_NCHUNK - 1].wait()


_sc_gather = functools.partial(
    pl.kernel,
    out_type=(
        jax.ShapeDtypeStruct((_BTS, _HALF), jnp.int32),
        jax.ShapeDtypeStruct((_BTS, _HALF), jnp.int32),
    ),
    mesh=plsc.VectorSubcoreMesh(core_axis_name="c", subcore_axis_name="s"),
    scratch_types=[
        pltpu.VMEM((_BPW,), jnp.int32),
        pltpu.VMEM((_BPW,), jnp.int32),
        pltpu.VMEM((_CH, _EMBED), jnp.float32),
        pltpu.VMEM((_CH, _EMBED), jnp.float32),
        pltpu.VMEM((_CH, _EMBED), jnp.float32),
        pltpu.VMEM((_CH, _EMBED), jnp.float32),
        pltpu.VMEM((_CH, _HALF), jnp.int32),
        pltpu.VMEM((_CH, _HALF), jnp.int32),
        pltpu.VMEM((_CH, _HALF), jnp.int32),
        pltpu.VMEM((_CH, _HALF), jnp.int32),
    ] + [pltpu.SemaphoreType.DMA] * 8,
)(_sc_gather_body)


_BLK = 8192


def _unpack_halves(v):
    f32 = jnp.float32
    lo = lax.bitcast_convert_type(jnp.left_shift(v, 16), f32)
    hi = lax.bitcast_convert_type(
        jnp.bitwise_and(v, jnp.int32(-65536)), f32)
    return lo, hi


def _tc_body(hot, trv, gcol, dcol, gtab, dtab,
             whp, bh, wtp, bt, wg, bg, wd, bd, out):
    f32 = jnp.float32
    h_lo, h_hi = _unpack_halves(hot[...])
    t_lo, t_hi = _unpack_halves(trv[...])
    hv = jnp.concatenate([h_lo, h_hi], axis=1)
    tv = jnp.concatenate([t_lo, t_hi], axis=1)
    h = jnp.tanh(jnp.dot(hv, whp[...], preferred_element_type=f32)
                 + bh[...])
    t = jnp.tanh(jnp.dot(tv, wtp[...], preferred_element_type=f32)
                 + bt[...])
    gp = jnp.tanh(jnp.dot(gtab[...], wg[...], preferred_element_type=f32)
                  + bg[...])
    dp = jnp.tanh(jnp.dot(dtab[...], wd[...], preferred_element_type=f32)
                  + bd[...])
    g0 = gp[0:1, :]
    dg = gp[1:2, :] - g0
    d0 = dp[0:1, :]
    dd = dp[1:2, :] - d0
    cmat = jnp.concatenate([g0 * d0, dg * d0, g0 * dd, dg * dd], axis=0)
    p = h * t
    qt = lax.dot_general(cmat, p, (((1,), (1,)), ((), ())),
                         preferred_element_type=f32)
    gf = gcol[0]
    df = dcol[0]
    out[0] = (qt[0:1, :] + gf * qt[1:2, :] + df * qt[2:3, :]
              + (gf * df) * qt[3:4, :])


def _tc_compute(hot, trv, gflat, dflat, gtab, dtab,
                whp, bh, wtp, bt, wg, bg, wd, bd):
    nblk = _BTS // _BLK
    grid = (nblk,)
    row_spec = pl.BlockSpec((_BLK, _HALF), lambda i: (i, 0))
    lane_spec = pl.BlockSpec((1, 1, _BLK), lambda i: (i, 0, 0))
    tab_spec = pl.BlockSpec((2, _EMBED), lambda i: (0, 0))
    w_spec = pl.BlockSpec((_EMBED, _PROJ), lambda i: (0, 0))
    b_spec = pl.BlockSpec((1, _PROJ), lambda i: (0, 0))
    return pl.pallas_call(
        _tc_body,
        grid=grid,
        in_specs=[row_spec, row_spec, lane_spec, lane_spec,
                  tab_spec, tab_spec,
                  w_spec, b_spec, w_spec, b_spec,
                  w_spec, b_spec, w_spec, b_spec],
        out_specs=lane_spec,
        out_shape=jax.ShapeDtypeStruct((nblk, 1, _BLK), jnp.float32),
    )(hot, trv, gflat, dflat, gtab, dtab,
      whp, bh, wtp, bt, wg, bg, wd, bd)


def kernel(hotel_id, travel_purpose, gender, desktop,
           hotel_table, travel_table, gender_table, device_table,
           W_h, b_h, W_t, b_t, W_g, b_g, W_d, b_d):
    nblk = _BT // _BLK
    nblk_s = _BTS // _BLK
    hid = hotel_id.reshape(_NSPLIT, _BTS).astype(jnp.int32)
    tid = travel_purpose.reshape(_NSPLIT, _BTS).astype(jnp.int32)
    gflat = gender.reshape(nblk, 1, _BLK).astype(jnp.float32)
    dflat = desktop.reshape(nblk, 1, _BLK).astype(jnp.float32)
    gathered = [_sc_gather(hotel_table, travel_table, hid[s], tid[s])
                for s in range(_NSPLIT)]
    w_arange = jnp.arange(_HALF)
    perm_lo = 32 * (w_arange // 16) + (w_arange % 16)
    perm = jnp.concatenate([perm_lo, perm_lo + 16])
    whp = W_h[perm, :]
    wtp = W_t[perm, :]
    bh, bt = b_h.reshape(1, _PROJ), b_t.reshape(1, _PROJ)
    bg, bd = b_g.reshape(1, _PROJ), b_d.reshape(1, _PROJ)
    outs = [
        _tc_compute(gathered[s][0], gathered[s][1],
                    gflat[s * nblk_s:(s + 1) * nblk_s],
                    dflat[s * nblk_s:(s + 1) * nblk_s],
                    gender_table, device_table,
                    whp, bh, wtp, bt, W_g, bg, W_d, bd)
        for s in range(_NSPLIT)
    ]
    return jnp.concatenate(outs, axis=0).reshape(_B, _L)


# speedup vs baseline: 1.6470x; 1.6470x over previous
import functools

import jax
import jax.numpy as jnp
from jax import lax
from jax.experimental import pallas as pl
from jax.experimental.pallas import tpu as pltpu
from jax.experimental.pallas import tpu_sc as plsc

_B, _L = 4096, 20
_BT = _B * _L
_EMBED = 128
_PROJ = 256
TRAVEL_ROWS = 1001

_NC, _NS = 2, 16
_NW = _NC * _NS
_NSPLIT = 2
_BTS = _BT // _NSPLIT
_BPW = _BTS // _NW
_CH = 160
_NCHUNK = _BPW // _CH


def _sc_gather_body(hotel_hbm, travel_hbm, hid_hbm, tid_hbm,
                    out_h_hbm, out_t_hbm,
                    idx_h, idx_t, trv_spmem, hbuf0, hbuf1, tbuf0, tbuf1,
                    gsh0, gsh1, gst0, gst1, wsh0, wsh1, wst0, wst1):
    hbufs, tbufs = (hbuf0, hbuf1), (tbuf0, tbuf1)
    gsems_h, gsems_t = (gsh0, gsh1), (gst0, gst1)
    wsems_h, wsems_t = (wsh0, wsh1), (wst0, wst1)
    wid = lax.axis_index("s") * _NC + lax.axis_index("c")
    base = wid * _BPW
    @pl.when(lax.axis_index("s") == 0)
    def _():
        pltpu.sync_copy(travel_hbm, trv_spmem)
    plsc.subcore_barrier()
    pltpu.sync_copy(hid_hbm.at[pl.ds(base, _BPW)], idx_h)
    pltpu.sync_copy(tid_hbm.at[pl.ds(base, _BPW)], idx_t)

    def g_h(c):
        return pltpu.async_copy(
            hotel_hbm.at[idx_h.at[pl.ds(c * _CH, _CH)]],
            hbufs[c % 2], gsems_h[c % 2])

    def g_t(c):
        return pltpu.async_copy(
            trv_spmem.at[idx_t.at[pl.ds(c * _CH, _CH)]],
            tbufs[c % 2], gsems_t[c % 2])

    def w_h(c):
        return pltpu.async_copy(
            hbufs[c % 2], out_h_hbm.at[pl.ds(base + c * _CH, _CH)],
            wsems_h[c % 2])

    def w_t(c):
        return pltpu.async_copy(
            tbufs[c % 2], out_t_hbm.at[pl.ds(base + c * _CH, _CH)],
            wsems_t[c % 2])

    gh = {0: g_h(0)}
    gt = {0: g_t(0)}
    wh, wt = {}, {}
    for c in range(_NCHUNK):
        if c >= 1:
            wh[c - 1].wait()
            wt[c - 1].wait()
        if c + 1 < _NCHUNK:
            gh[c + 1] = g_h(c + 1)
            gt[c + 1] = g_t(c + 1)
        gh[c].wait()
        wh[c] = w_h(c)
        gt[c].wait()
        wt[c] = w_t(c)
    wh[_NCHUNK - 1].wait()
    wt[_NCHUNK - 1].wait()


_sc_gather = functools.partial(
    pl.kernel,
    out_type=(
        jax.ShapeDtypeStruct((_BTS, _EMBED), jnp.float32),
        jax.ShapeDtypeStruct((_BTS, _EMBED), jnp.float32),
    ),
    mesh=plsc.VectorSubcoreMesh(core_axis_name="c", subcore_axis_name="s"),
    scratch_types=[
        pltpu.VMEM((_BPW,), jnp.int32),
        pltpu.VMEM((_BPW,), jnp.int32),
        pltpu.VMEM_SHARED((1001, _EMBED), jnp.float32),
        pltpu.VMEM((_CH, _EMBED), jnp.float32),
        pltpu.VMEM((_CH, _EMBED), jnp.float32),
        pltpu.VMEM((_CH, _EMBED), jnp.float32),
        pltpu.VMEM((_CH, _EMBED), jnp.float32),
    ] + [pltpu.SemaphoreType.DMA] * 8,
)(_sc_gather_body)


_BLK = 8192


def _tc_body(hot, trv, gcol, dcol, gtab, dtab,
             wh, bh, wt, bt, wg, bg, wd, bd, out):
    f32 = jnp.float32
    h = jnp.tanh(jnp.dot(hot[...], wh[...], preferred_element_type=f32)
                 + bh[...])
    t = jnp.tanh(jnp.dot(trv[...], wt[...], preferred_element_type=f32)
                 + bt[...])
    gp = jnp.tanh(jnp.dot(gtab[...], wg[...], preferred_element_type=f32)
                  + bg[...])
    dp = jnp.tanh(jnp.dot(dtab[...], wd[...], preferred_element_type=f32)
                  + bd[...])
    g0 = gp[0:1, :]
    dg = gp[1:2, :] - g0
    d0 = dp[0:1, :]
    dd = dp[1:2, :] - d0
    cmat = jnp.concatenate([g0 * d0, dg * d0, g0 * dd, dg * dd], axis=0)
    p = h * t
    qt = lax.dot_general(cmat, p, (((1,), (1,)), ((), ())),
                         preferred_element_type=f32)
    gf = gcol[0]
    df = dcol[0]
    out[0] = (qt[0:1, :] + gf * qt[1:2, :] + df * qt[2:3, :]
              + (gf * df) * qt[3:4, :])


def _tc_compute(hot, trv, gflat, dflat, gtab, dtab,
                wh, bh, wt, bt, wg, bg, wd, bd):
    nblk = _BTS // _BLK
    grid = (nblk,)
    row_spec = pl.BlockSpec((_BLK, _EMBED), lambda i: (i, 0))
    bf_spec = pl.BlockSpec((_BLK, _EMBED), lambda i: (i, 0))
    lane_spec = pl.BlockSpec((1, 1, _BLK), lambda i: (i, 0, 0))
    tab_spec = pl.BlockSpec((2, _EMBED), lambda i: (0, 0))
    w_spec = pl.BlockSpec((_EMBED, _PROJ), lambda i: (0, 0))
    b_spec = pl.BlockSpec((1, _PROJ), lambda i: (0, 0))
    return pl.pallas_call(
        _tc_body,
        grid=grid,
        in_specs=[row_spec, bf_spec, lane_spec, lane_spec,
                  tab_spec, tab_spec,
                  w_spec, b_spec, w_spec, b_spec,
                  w_spec, b_spec, w_spec, b_spec],
        out_specs=lane_spec,
        out_shape=jax.ShapeDtypeStruct((nblk, 1, _BLK), jnp.float32),
    )(hot, trv, gflat, dflat, gtab, dtab,
      wh, bh, wt, bt, wg, bg, wd, bd)


def kernel(hotel_id, travel_purpose, gender, desktop,
           hotel_table, travel_table, gender_table, device_table,
           W_h, b_h, W_t, b_t, W_g, b_g, W_d, b_d):
    nblk = _BT // _BLK
    nblk_s = _BTS // _BLK
    hid = hotel_id.reshape(_NSPLIT, _BTS).astype(jnp.int32)
    tid = travel_purpose.reshape(_NSPLIT, _BTS).astype(jnp.int32)
    gflat = gender.reshape(nblk, 1, _BLK).astype(jnp.float32)
    dflat = desktop.reshape(nblk, 1, _BLK).astype(jnp.float32)
    gathered = [_sc_gather(hotel_table, travel_table, hid[s], tid[s])
                for s in range(_NSPLIT)]
    bh, bt = b_h.reshape(1, _PROJ), b_t.reshape(1, _PROJ)
    bg, bd = b_g.reshape(1, _PROJ), b_d.reshape(1, _PROJ)
    outs = [
        _tc_compute(gathered[s][0], gathered[s][1],
                    gflat[s * nblk_s:(s + 1) * nblk_s],
                    dflat[s * nblk_s:(s + 1) * nblk_s],
                    gender_table, device_table,
                    W_h, bh, W_t, bt, W_g, bg, W_d, bd)
        for s in range(_NSPLIT)
    ]
    return jnp.concatenate(outs, axis=0).reshape(_B, _L)


# speedup vs baseline: 1.6612x; 1.0086x over previous
import functools

import jax
import jax.numpy as jnp
from jax import lax
from jax.experimental import pallas as pl
from jax.experimental.pallas import tpu as pltpu
from jax.experimental.pallas import tpu_sc as plsc

_B, _L = 4096, 20
_BT = _B * _L
_EMBED = 128
_PROJ = 256
TRAVEL_ROWS = 1001

_NC, _NS = 2, 16
_NW = _NC * _NS
_NSPLIT = 2
_BTS = _BT // _NSPLIT
_BPW = _BTS // _NW
_CH = 160
_NCHUNK = _BPW // _CH


def _sc_gather_body(hotel_hbm, travel_hbm, hid_hbm, tid_hbm,
                    out_h_hbm, out_t_hbm,
                    idx_h, idx_t, trv_spmem, hbuf0, hbuf1, tbuf0, tbuf1,
                    gsh0, gsh1, gst0, gst1, wsh0, wsh1, wst0, wst1):
    hbufs, tbufs = (hbuf0, hbuf1), (tbuf0, tbuf1)
    gsems_h, gsems_t = (gsh0, gsh1), (gst0, gst1)
    wsems_h, wsems_t = (wsh0, wsh1), (wst0, wst1)
    wid = lax.axis_index("s") * _NC + lax.axis_index("c")
    base = wid * _BPW
    @pl.when(lax.axis_index("s") == 0)
    def _():
        pltpu.sync_copy(travel_hbm, trv_spmem)
    plsc.subcore_barrier()
    pltpu.sync_copy(hid_hbm.at[pl.ds(base, _BPW)], idx_h)
    pltpu.sync_copy(tid_hbm.at[pl.ds(base, _BPW)], idx_t)

    def g_h(c):
        return pltpu.async_copy(
            hotel_hbm.at[idx_h.at[pl.ds(c * _CH, _CH)]],
            hbufs[c % 2], gsems_h[c % 2])

    def g_t(c):
        return pltpu.async_copy(
            trv_spmem.at[idx_t.at[pl.ds(c * _CH, _CH)]],
            tbufs[c % 2], gsems_t[c % 2])

    def w_h(c):
        return pltpu.async_copy(
            hbufs[c % 2], out_h_hbm.at[pl.ds(base + c * _CH, _CH)],
            wsems_h[c % 2])

    def w_t(c):
        return pltpu.async_copy(
            tbufs[c % 2], out_t_hbm.at[pl.ds(base + c * _CH, _CH)],
            wsems_t[c % 2])

    gh = {0: g_h(0)}
    gt = {0: g_t(0)}
    wh, wt = {}, {}
    for c in range(_NCHUNK):
        if c >= 1:
            wh[c - 1].wait()
            wt[c - 1].wait()
        if c + 1 < _NCHUNK:
            gh[c + 1] = g_h(c + 1)
            gt[c + 1] = g_t(c + 1)
        gh[c].wait()
        wh[c] = w_h(c)
        gt[c].wait()
        wt[c] = w_t(c)
    wh[_NCHUNK - 1].wait()
    wt[_NCHUNK - 1].wait()


_sc_gather = functools.partial(
    pl.kernel,
    out_type=(
        jax.ShapeDtypeStruct((_BTS, _EMBED), jnp.float32),
        jax.ShapeDtypeStruct((_BTS, _EMBED), jnp.float32),
    ),
    mesh=plsc.VectorSubcoreMesh(core_axis_name="c", subcore_axis_name="s"),
    scratch_types=[
        pltpu.VMEM((_BPW,), jnp.int32),
        pltpu.VMEM((_BPW,), jnp.int32),
        pltpu.VMEM_SHARED((1001, _EMBED), jnp.float32),
        pltpu.VMEM((_CH, _EMBED), jnp.float32),
        pltpu.VMEM((_CH, _EMBED), jnp.float32),
        pltpu.VMEM((_CH, _EMBED), jnp.float32),
        pltpu.VMEM((_CH, _EMBED), jnp.float32),
    ] + [pltpu.SemaphoreType.DMA] * 8,
)(_sc_gather_body)


_BLK = 8192


def _tc_body(hot, trv, gcol, dcol, gtab, dtab,
             wh, bh, wt, bt, wg, bg, wd, bd, out):
    f32 = jnp.float32
    h = jnp.tanh(jnp.dot(hot[...], wh[...], preferred_element_type=f32)
                 + bh[...])
    t = jnp.tanh(jnp.dot(trv[...], wt[...], preferred_element_type=f32)
                 + bt[...])
    gp = jnp.tanh(jnp.dot(gtab[...], wg[...], preferred_element_type=f32)
                  + bg[...])
    dp = jnp.tanh(jnp.dot(dtab[...], wd[...], preferred_element_type=f32)
                  + bd[...])
    g0 = gp[0:1, :]
    dg = gp[1:2, :] - g0
    d0 = dp[0:1, :]
    dd = dp[1:2, :] - d0
    cmat = jnp.concatenate([g0 * d0, dg * d0, g0 * dd, dg * dd], axis=0)
    p = h * t
    qt = lax.dot_general(cmat, p, (((1,), (1,)), ((), ())),
                         preferred_element_type=f32)
    gf = gcol[0]
    df = dcol[0]
    out[0] = (qt[0:1, :] + gf * qt[1:2, :] + df * qt[2:3, :]
              + (gf * df) * qt[3:4, :])


def _tc_compute(hot, trv, gflat, dflat, gtab, dtab,
                wh, bh, wt, bt, wg, bg, wd, bd):
    nblk = _BTS // _BLK
    grid = (nblk,)
    row_spec = pl.BlockSpec((_BLK, _EMBED), lambda i: (i, 0))
    bf_spec = pl.BlockSpec((_BLK, _EMBED), lambda i: (i, 0))
    lane_spec = pl.BlockSpec((1, 1, _BLK), lambda i: (i, 0, 0))
    tab_spec = pl.BlockSpec((2, _EMBED), lambda i: (0, 0))
    w_spec = pl.BlockSpec((_EMBED, _PROJ), lambda i: (0, 0))
    b_spec = pl.BlockSpec((1, _PROJ), lambda i: (0, 0))
    return pl.pallas_call(
        _tc_body,
        grid=grid,
        in_specs=[row_spec, bf_spec, lane_spec, lane_spec,
                  tab_spec, tab_spec,
                  w_spec, b_spec, w_spec, b_spec,
                  w_spec, b_spec, w_spec, b_spec],
        out_specs=lane_spec,
        out_shape=jax.ShapeDtypeStruct((nblk, 1, _BLK), jnp.float32),
    )(hot, trv, gflat, dflat, gtab, dtab,
      wh, bh, wt, bt, wg, bg, wd, bd)


def kernel(hotel_id, travel_purpose, gender, desktop,
           hotel_table, travel_table, gender_table, device_table,
           W_h, b_h, W_t, b_t, W_g, b_g, W_d, b_d):
    nblk = _BT // _BLK
    nblk_s = _BTS // _BLK
    rows_s = _B // _NSPLIT
    hid = [hotel_id[s * rows_s:(s + 1) * rows_s].reshape(_BTS)
           .astype(jnp.int32) for s in range(_NSPLIT)]
    tid = [travel_purpose[s * rows_s:(s + 1) * rows_s].reshape(_BTS)
           .astype(jnp.int32) for s in range(_NSPLIT)]
    gflat = gender.reshape(nblk, 1, _BLK).astype(jnp.float32)
    dflat = desktop.reshape(nblk, 1, _BLK).astype(jnp.float32)
    gathered = [_sc_gather(hotel_table, travel_table, hid[s], tid[s])
                for s in range(_NSPLIT)]
    bh, bt = b_h.reshape(1, _PROJ), b_t.reshape(1, _PROJ)
    bg, bd = b_g.reshape(1, _PROJ), b_d.reshape(1, _PROJ)
    outs = [
        _tc_compute(gathered[s][0], gathered[s][1],
                    gflat[s * nblk_s:(s + 1) * nblk_s],
                    dflat[s * nblk_s:(s + 1) * nblk_s],
                    gender_table, device_table,
                    W_h, bh, W_t, bt, W_g, bg, W_d, bd)
        for s in range(_NSPLIT)
    ]
    return jnp.concatenate(outs, axis=0).reshape(_B, _L)
